# Initial kernel scaffold; baseline (speedup 1.0000x reference)
import jax
import jax.numpy as jnp
from jax.experimental import pallas as pl


def kernel(x, enc_Wih0, enc_Whh0, enc_bih0, enc_bhh0, enc_Wih1, enc_Whh1, enc_bih1, enc_bhh1, dec_Wih0, dec_Whh0, dec_bih0, dec_bhh0, dec_Wih1, dec_Whh1, dec_bih1, dec_bhh1, fc_w, fc_b, cluster):
    raise NotImplementedError("write your pallas kernel here")


# speedup vs baseline: 3.4746x; 3.4746x over previous
import jax
import jax.numpy as jnp
from jax.experimental import pallas as pl
from jax.experimental.pallas import tpu as pltpu

B, T, I, H, K = 512, 100, 128, 256, 8


def _gru_cell(x_bf, h_prev, wih_ref, whh_ref, brz_ref, bin_ref, bhn_ref, hd):
    gi = jnp.dot(x_bf, wih_ref[...], preferred_element_type=jnp.float32)
    gh = jnp.dot(h_prev.astype(jnp.bfloat16), whh_ref[...],
                 preferred_element_type=jnp.float32)
    rz = jax.nn.sigmoid(gi[:, : 2 * hd] + gh[:, : 2 * hd] + brz_ref[...])
    r = rz[:, :hd]
    zg = rz[:, hd:]
    n = jnp.tanh(gi[:, 2 * hd:] + bin_ref[...] + r * (gh[:, 2 * hd:] + bhn_ref[...]))
    return (1.0 - zg) * n + zg * h_prev


def _fused_kernel(x_ref, c_ref,
                  wih0, whh0, brz0, bin0, bhn0,
                  wih1, whh1, brz1, bin1, bhn1,
                  wih2, whh2, brz2, bin2, bhn2,
                  wih3, whh3, brz3, bin3, bhn3,
                  xbar_ref, z_ref, q_ref,
                  h1, h2, h3, h4):
    t = pl.program_id(0)

    @pl.when(t == 0)
    def _init():
        h1[...] = jnp.zeros_like(h1)
        h2[...] = jnp.zeros_like(h2)
        h3[...] = jnp.zeros_like(h3)
        h4[...] = jnp.zeros_like(h4)

    x_t = x_ref[...]
    nh1 = _gru_cell(x_t, h1[...], wih0, whh0, brz0, bin0, bhn0, H)
    h1[...] = nh1
    nh2 = _gru_cell(nh1.astype(jnp.bfloat16), h2[...], wih1, whh1, brz1, bin1, bhn1, H)
    h2[...] = nh2
    nh3 = _gru_cell(nh2.astype(jnp.bfloat16), h3[...], wih2, whh2, brz2, bin2, bhn2, I)
    h3[...] = nh3
    nh4 = _gru_cell(nh3.astype(jnp.bfloat16), h4[...], wih3, whh3, brz3, bin3, bhn3, I)
    h4[...] = nh4
    xbar_ref[...] = nh4

    @pl.when(t == T - 1)
    def _final():
        z = nh2
        z_ref[...] = z
        cols = []
        for k in range(K):
            d = z - c_ref[k, :]
            cols.append(jnp.sum(d * d, axis=1, keepdims=True))
        d2 = jnp.concatenate(cols, axis=1)
        qu = 1.0 / (1.0 + d2)
        q_ref[...] = qu / jnp.sum(qu, axis=1, keepdims=True)


def _prep_layer(Wih, Whh, bih, bhh, hd):
    wih = Wih.T.astype(jnp.bfloat16)
    whh = Whh.T.astype(jnp.bfloat16)
    brz = (bih[: 2 * hd] + bhh[: 2 * hd]).reshape(1, 2 * hd)
    bin_ = bih[2 * hd:].reshape(1, hd)
    bhn = bhh[2 * hd:].reshape(1, hd)
    return wih, whh, brz, bin_, bhn


def kernel(x, enc_Wih0, enc_Whh0, enc_bih0, enc_bhh0, enc_Wih1, enc_Whh1,
           enc_bih1, enc_bhh1, dec_Wih0, dec_Whh0, dec_bih0, dec_bhh0,
           dec_Wih1, dec_Whh1, dec_bih1, dec_bhh1, fc_w, fc_b, cluster):
    del fc_w, fc_b
    l0 = _prep_layer(enc_Wih0, enc_Whh0, enc_bih0, enc_bhh0, H)
    l1 = _prep_layer(enc_Wih1, enc_Whh1, enc_bih1, enc_bhh1, H)
    l2 = _prep_layer(dec_Wih0, dec_Whh0, dec_bih0, dec_bhh0, I)
    l3 = _prep_layer(dec_Wih1, dec_Whh1, dec_bih1, dec_bhh1, I)

    x2 = x.reshape(B, T * I).astype(jnp.bfloat16)

    def whole(shape):
        return pl.BlockSpec(shape, lambda t: (0, 0))

    in_specs = [
        pl.BlockSpec((B, I), lambda t: (0, t)),
        whole((K, H)),
    ]
    for (wih, whh, brz, bin_, bhn) in (l0, l1, l2, l3):
        in_specs += [whole(wih.shape), whole(whh.shape), whole(brz.shape),
                     whole(bin_.shape), whole(bhn.shape)]

    out_specs = [
        pl.BlockSpec((B, I), lambda t: (0, t)),
        whole((B, H)),
        whole((B, K)),
    ]
    out_shape = [
        jax.ShapeDtypeStruct((B, T * I), jnp.float32),
        jax.ShapeDtypeStruct((B, H), jnp.float32),
        jax.ShapeDtypeStruct((B, K), jnp.float32),
    ]

    xbar, z, q = pl.pallas_call(
        _fused_kernel,
        grid=(T,),
        in_specs=in_specs,
        out_specs=out_specs,
        out_shape=out_shape,
        scratch_shapes=[
            pltpu.VMEM((B, H), jnp.float32),
            pltpu.VMEM((B, H), jnp.float32),
            pltpu.VMEM((B, I), jnp.float32),
            pltpu.VMEM((B, I), jnp.float32),
        ],
        compiler_params=pltpu.CompilerParams(
            dimension_semantics=("arbitrary",),
        ),
    )(x2, cluster, *l0, *l1, *l2, *l3)

    return (z, xbar.reshape(B, T, I), q)
